# Initial kernel scaffold; baseline (speedup 1.0000x reference)
import jax
import jax.numpy as jnp
from jax.experimental import pallas as pl


def kernel(x):
    raise NotImplementedError("write your pallas kernel here")


# speedup vs baseline: 134.1937x; 134.1937x over previous
import jax
import jax.numpy as jnp
from jax.experimental import pallas as pl

_B = 4096
_K = 2048
_W = 256


def _topk_mask_kernel(x_ref, o_ref):
    x = x_ref[...]
    i = jax.lax.bitcast_convert_type(x, jnp.int32)
    flip = jax.lax.shift_right_arithmetic(i, 31) & jnp.int32(0x7FFFFFFF)
    y = i ^ flip
    z = jax.lax.bitcast_convert_type(y, jnp.uint32) ^ jnp.uint32(0x80000000)

    def body(t, prefix):
        shift = (jnp.uint32(31) - t.astype(jnp.uint32))
        bit = jax.lax.shift_left(jnp.uint32(1), shift)
        cand = prefix | bit
        cnt = jnp.sum((z >= cand).astype(jnp.int32), axis=0, keepdims=True)
        return jnp.where(cnt >= _K, cand, prefix)

    thr = jax.lax.fori_loop(0, 32, body, jnp.zeros((1, _W), jnp.uint32))
    keep = z >= thr
    o_ref[...] = jnp.where(keep, x, 0.0)


def kernel(x):
    grid = (x.shape[1] // _W,)
    return pl.pallas_call(
        _topk_mask_kernel,
        grid=grid,
        in_specs=[pl.BlockSpec((_B, _W), lambda j: (0, j))],
        out_specs=pl.BlockSpec((_B, _W), lambda j: (0, j)),
        out_shape=jax.ShapeDtypeStruct(x.shape, x.dtype),
    )(x)


# speedup vs baseline: 238.7372x; 1.7790x over previous
import jax
import jax.numpy as jnp
from jax import lax
from jax.experimental import pallas as pl

_B = 4096
_K = 2048
_W = 256


def _count_ge(key, cand, strict=False):
    def rec(lo, hi):
        if hi - lo == 16:
            m = (key[lo:hi] > cand) if strict else (key[lo:hi] >= cand)
            return jnp.where(m, jnp.int16(1), jnp.int16(0))
        mid = (lo + hi) // 2
        return rec(lo, mid) + rec(mid, hi)
    s = rec(0, key.shape[0])
    return jnp.sum(s.astype(jnp.int32), axis=0, keepdims=True)


def _topk_mask_kernel(x_ref, o_ref):
    x = x_ref[...]
    i = lax.bitcast_convert_type(x, jnp.int32)
    flip = lax.shift_right_arithmetic(i, 31) & jnp.int32(0x7FFFFFFF)
    y = i ^ flip
    z = lax.bitcast_convert_type(y, jnp.uint32) ^ jnp.uint32(0x80000000)

    hi32 = (z >> jnp.uint32(16)).astype(jnp.int32)
    hs = (hi32 - 32768).astype(jnp.int16)
    p = jnp.zeros((1, _W), jnp.int32)
    for bit in range(15, -1, -1):
        cand = p | (1 << bit)
        ok = _count_ge(hs, (cand - 32768).astype(jnp.int16)) >= _K
        p = jnp.where(ok, cand, p)

    ps = (p - 32768).astype(jnp.int16)
    budget = _K - _count_ge(hs, ps, strict=True)
    lo32 = (z & jnp.uint32(0xFFFF)).astype(jnp.int32)
    los = jnp.where(hs == ps, (lo32 - 32768).astype(jnp.int16),
                    jnp.int16(-32768))
    q = jnp.zeros((1, _W), jnp.int32)
    for bit in range(15, -1, -1):
        cand = q | (1 << bit)
        ok = _count_ge(los, (cand - 32768).astype(jnp.int16)) >= budget
        q = jnp.where(ok, cand, q)

    thr = lax.bitcast_convert_type((p << 16) | q, jnp.uint32)
    keep = z >= thr
    o_ref[...] = jnp.where(keep, x, 0.0)


def kernel(x):
    grid = (x.shape[1] // _W,)
    return pl.pallas_call(
        _topk_mask_kernel,
        grid=grid,
        in_specs=[pl.BlockSpec((_B, _W), lambda j: (0, j))],
        out_specs=pl.BlockSpec((_B, _W), lambda j: (0, j)),
        out_shape=jax.ShapeDtypeStruct(x.shape, x.dtype),
    )(x)
